# Initial kernel scaffold; baseline (speedup 1.0000x reference)
import jax
import jax.numpy as jnp
from jax.experimental import pallas as pl


def kernel(program, concept_table, relation_table):
    raise NotImplementedError("write your pallas kernel here")


# speedup vs baseline: 1.6708x; 1.6708x over previous
import functools

import jax
import jax.numpy as jnp
from jax import lax
from jax.experimental import pallas as pl
from jax.experimental.pallas import tpu as pltpu
from jax.experimental.pallas import tpu_sc as plsc

_EMBED = 32
_NC = 2
_NS = 16
_NW = _NC * _NS
_CHUNK = 128


def _make_gather2(B):
    bpw = B // _NW
    nchunk = bpw // _CHUNK
    mesh = plsc.VectorSubcoreMesh(core_axis_name="c", subcore_axis_name="s")

    @functools.partial(
        pl.kernel,
        mesh=mesh,
        compiler_params=pltpu.CompilerParams(use_tc_tiling_on_sc=False),
        out_type=jax.ShapeDtypeStruct((B, 2, _EMBED), jnp.float32),
        scratch_types=[
            pltpu.VMEM((nchunk, _CHUNK), jnp.int32),
            pltpu.VMEM((nchunk, _CHUNK, _EMBED), jnp.float32),
            pltpu.VMEM((nchunk, _CHUNK, _EMBED), jnp.float32),
            pltpu.SemaphoreType.DMA,
            pltpu.SemaphoreType.DMA,
            pltpu.SemaphoreType.DMA,
        ],
    )
    def gather2(idx_hbm, ct_hbm, rt_hbm, out_hbm,
                idx_v, rows_c, rows_r, sem_c, sem_r, sem_w):
        wid = lax.axis_index("s") * _NC + lax.axis_index("c")
        base = wid * bpw
        pltpu.sync_copy(idx_hbm.at[wid], idx_v)
        gc = [pltpu.async_copy(ct_hbm.at[idx_v.at[j]], rows_c.at[j], sem_c)
              for j in range(nchunk)]
        gr = [pltpu.async_copy(rt_hbm.at[idx_v.at[j]], rows_r.at[j], sem_r)
              for j in range(nchunk)]
        wr = []
        for j in range(nchunk):
            gc[j].wait()
            wr.append(pltpu.async_copy(
                rows_c.at[j], out_hbm.at[pl.ds(base + j * _CHUNK, _CHUNK), 0],
                sem_w))
            gr[j].wait()
            wr.append(pltpu.async_copy(
                rows_r.at[j], out_hbm.at[pl.ds(base + j * _CHUNK, _CHUNK), 1],
                sem_w))
        for w in wr:
            w.wait()

    return gather2


_B = 1024 * 20
_GATHER2 = _make_gather2(_B)


def kernel(program, concept_table, relation_table):
    batch, prog_len = program.shape[0], program.shape[1]
    idx = program[:, :, 1].astype(jnp.int32).reshape(_NW, -1, _CHUNK)
    out = _GATHER2(idx, concept_table, relation_table)
    out = out.reshape(batch, prog_len, 2 * _EMBED)
    return out, concept_table


# speedup vs baseline: 1.7092x; 1.0230x over previous
import functools

import jax
import jax.numpy as jnp
from jax import lax
from jax.experimental import pallas as pl
from jax.experimental.pallas import tpu as pltpu
from jax.experimental.pallas import tpu_sc as plsc

_EMBED = 32
_NC = 2
_NS = 16
_NW = _NC * _NS
_CHUNK = 128


def _make_gather2(B):
    bpw = B // _NW
    nchunk = bpw // _CHUNK
    mesh = plsc.VectorSubcoreMesh(core_axis_name="c", subcore_axis_name="s")

    @functools.partial(
        pl.kernel,
        mesh=mesh,
        compiler_params=pltpu.CompilerParams(use_tc_tiling_on_sc=False),
        out_type=jax.ShapeDtypeStruct((B, 2, _EMBED), jnp.float32),
        scratch_types=[
            pltpu.VMEM((nchunk, _CHUNK), jnp.int32),
            pltpu.VMEM((nchunk, _CHUNK, _EMBED), jnp.float32),
            pltpu.VMEM((nchunk, _CHUNK, _EMBED), jnp.float32),
            pltpu.SemaphoreType.DMA,
            pltpu.SemaphoreType.DMA,
            pltpu.SemaphoreType.DMA,
        ],
    )
    def gather2(idx_hbm, ct_hbm, rt_hbm, out_hbm,
                idx_v, rows_c, rows_r, sem_c, sem_r, sem_w):
        wid = lax.axis_index("s") * _NC + lax.axis_index("c")
        base = wid * bpw
        pltpu.sync_copy(idx_hbm.at[wid], idx_v)
        gc = [pltpu.async_copy(ct_hbm.at[idx_v.at[j]], rows_c.at[j], sem_c)
              for j in range(nchunk)]
        gr = [pltpu.async_copy(rt_hbm.at[idx_v.at[j]], rows_r.at[j], sem_r)
              for j in range(nchunk)]
        wr = []
        for j in range(nchunk):
            gc[j].wait()
            wr.append(pltpu.async_copy(
                rows_c.at[j], out_hbm.at[pl.ds(base + j * _CHUNK, _CHUNK), 0],
                sem_w))
            gr[j].wait()
            wr.append(pltpu.async_copy(
                rows_r.at[j], out_hbm.at[pl.ds(base + j * _CHUNK, _CHUNK), 1],
                sem_w))
        for w in wr:
            w.wait()

    return gather2


_B = 1024 * 20
_GATHER2 = _make_gather2(_B)


def _tc_copy_kernel(in_ref, out_ref):
    out_ref[...] = in_ref[...]


def _tc_copy_t(table_t):
    d, v = table_t.shape
    blk = 8
    return pl.pallas_call(
        _tc_copy_kernel,
        grid=(d // blk,),
        in_specs=[pl.BlockSpec((blk, v), lambda i: (i, 0))],
        out_specs=pl.BlockSpec((blk, v), lambda i: (i, 0)),
        out_shape=jax.ShapeDtypeStruct((d, v), table_t.dtype),
    )(table_t)


def kernel(program, concept_table, relation_table):
    batch, prog_len = program.shape[0], program.shape[1]
    idx = program[:, :, 1].astype(jnp.int32).reshape(_NW, -1, _CHUNK)
    out = _GATHER2(idx, concept_table, relation_table)
    out = out.reshape(batch, prog_len, 2 * _EMBED)
    all_concepts = _tc_copy_t(concept_table.T).T
    return out, all_concepts
